# baseline (speedup 1.0000x reference)
import functools
import jax
import jax.numpy as jnp
from jax import lax
from jax.experimental import pallas as pl
from jax.experimental.pallas import tpu as pltpu

NA = 2048
NB = 2048
K = 30
BIG = 3.0e38

INTERPRET = False
WA_BR = 1000
EMB_BR = 512
KNN_BR = 256


def _wa_body(gm_ref, wtop_ref, wbot_ref, out_ref):
    gm = gm_ref[...]
    rs = jnp.sum(gm, axis=1, keepdims=True)
    rs = jnp.where(rs == 0.0, 1.0, rs)
    acc = jax.lax.dot_general(gm, wbot_ref[...], (((1,), (0,)), ((), ())),
                              preferred_element_type=jnp.float32)
    out_ref[...] = wtop_ref[...] + acc / rs


def _wa_call(gm_a, w_top, w_bot):
    n, m = gm_a.shape
    br = WA_BR
    grid = (n // br,)
    return pl.pallas_call(
        _wa_body,
        grid=grid,
        in_specs=[
            pl.BlockSpec((br, m), lambda i: (i, 0)),
            pl.BlockSpec((br, 32), lambda i: (i, 0)),
            pl.BlockSpec((m, 32), lambda i: (0, 0)),
        ],
        out_specs=pl.BlockSpec((br, 32), lambda i: (i, 0)),
        out_shape=jax.ShapeDtypeStruct((n, 32), jnp.float32),
        interpret=INTERPRET,
    )(gm_a, w_top, w_bot)


def _wb_body(gm_ref, wtop_ref, z_ref, cs_ref):
    p = pl.program_id(0)
    gm = gm_ref[...]
    z_part = jax.lax.dot_general(gm, wtop_ref[...], (((0,), (0,)), ((), ())),
                                 preferred_element_type=jnp.float32)
    cs_part = jnp.sum(gm, axis=0, keepdims=True)

    @pl.when(p == 0)
    def _():
        z_ref[...] = z_part
        cs_ref[...] = cs_part

    @pl.when(p != 0)
    def _():
        z_ref[...] = z_ref[...] + z_part
        cs_ref[...] = cs_ref[...] + cs_part


def _wb_call(gm_b, w_top):
    n, m = gm_b.shape
    br = WA_BR
    grid = (n // br,)
    return pl.pallas_call(
        _wb_body,
        grid=grid,
        in_specs=[
            pl.BlockSpec((br, m), lambda i: (i, 0)),
            pl.BlockSpec((br, 32), lambda i: (i, 0)),
        ],
        out_specs=[
            pl.BlockSpec((m, 32), lambda i: (0, 0)),
            pl.BlockSpec((1, m), lambda i: (0, 0)),
        ],
        out_shape=[
            jax.ShapeDtypeStruct((m, 32), jnp.float32),
            jax.ShapeDtypeStruct((1, m), jnp.float32),
        ],
        interpret=INTERPRET,
    )(gm_b, w_top)


def _cls_tail(e, w1_ref, b1_ref, w2_ref, b2_ref):
    h = jax.nn.relu(jnp.dot(e, w1_ref[...], preferred_element_type=jnp.float32)
                    + b1_ref[...][None, :])
    logit = jnp.dot(h, w2_ref[...], preferred_element_type=jnp.float32) \
        + b2_ref[...][None, :]
    mx = jnp.max(logit, axis=1, keepdims=True)
    ex = jnp.exp(logit - mx)
    return ex / jnp.sum(ex, axis=1, keepdims=True)


def _embA_body(x_ref, mean_ref, std_ref, w_ref, b_ref,
               w1_ref, b1_ref, w2_ref, b2_ref, xa_ref, pred_ref):
    xn = (x_ref[...] - mean_ref[...][None, :]) / std_ref[...][None, :]
    e = jax.nn.relu(jnp.dot(xn, w_ref[...], preferred_element_type=jnp.float32)
                    + b_ref[...][None, :])
    xa_ref[...] = e
    pred_ref[...] = _cls_tail(e, w1_ref, b1_ref, w2_ref, b2_ref)


def _embA_call(x, mean, std, w_eff, b, w1, b1, w2, b2, n_cls):
    nr, d = x.shape
    br = EMB_BR
    grid = (nr // br,)
    return pl.pallas_call(
        _embA_body,
        grid=grid,
        in_specs=[
            pl.BlockSpec((br, d), lambda i: (i, 0)),
            pl.BlockSpec((d,), lambda i: (0,)),
            pl.BlockSpec((d,), lambda i: (0,)),
            pl.BlockSpec((d, 32), lambda i: (0, 0)),
            pl.BlockSpec((32,), lambda i: (0,)),
            pl.BlockSpec((32, 16), lambda i: (0, 0)),
            pl.BlockSpec((16,), lambda i: (0,)),
            pl.BlockSpec((16, n_cls), lambda i: (0, 0)),
            pl.BlockSpec((n_cls,), lambda i: (0,)),
        ],
        out_specs=[
            pl.BlockSpec((br, 32), lambda i: (i, 0)),
            pl.BlockSpec((br, n_cls), lambda i: (i, 0)),
        ],
        out_shape=[
            jax.ShapeDtypeStruct((nr, 32), jnp.float32),
            jax.ShapeDtypeStruct((nr, n_cls), jnp.float32),
        ],
        interpret=INTERPRET,
    )(x, mean, std, w_eff, b, w1, b1, w2, b2)


def _embB_body(x_ref, mean_ref, std_ref, cs_ref, z_ref, wbot_ref, b_ref,
               w1_ref, b1_ref, w2_ref, b2_ref, xb_ref, pred_ref):
    xn = (x_ref[...] - mean_ref[...][None, :]) / std_ref[...][None, :]
    cs = cs_ref[...]
    cs = jnp.where(cs == 0.0, 1.0, cs)
    xs = xn / cs
    e = jnp.dot(xs, z_ref[...], preferred_element_type=jnp.float32) \
        + jnp.dot(xn, wbot_ref[...], preferred_element_type=jnp.float32) \
        + b_ref[...][None, :]
    e = jax.nn.relu(e)
    xb_ref[...] = e
    pred_ref[...] = _cls_tail(e, w1_ref, b1_ref, w2_ref, b2_ref)


def _embB_call(x, mean, std, cs, z, w_bot, b, w1, b1, w2, b2, n_cls):
    nr, d = x.shape
    br = EMB_BR
    grid = (nr // br,)
    return pl.pallas_call(
        _embB_body,
        grid=grid,
        in_specs=[
            pl.BlockSpec((br, d), lambda i: (i, 0)),
            pl.BlockSpec((d,), lambda i: (0,)),
            pl.BlockSpec((d,), lambda i: (0,)),
            pl.BlockSpec((1, d), lambda i: (0, 0)),
            pl.BlockSpec((d, 32), lambda i: (0, 0)),
            pl.BlockSpec((d, 32), lambda i: (0, 0)),
            pl.BlockSpec((32,), lambda i: (0,)),
            pl.BlockSpec((32, 16), lambda i: (0, 0)),
            pl.BlockSpec((16,), lambda i: (0,)),
            pl.BlockSpec((16, n_cls), lambda i: (0, 0)),
            pl.BlockSpec((n_cls,), lambda i: (0,)),
        ],
        out_specs=[
            pl.BlockSpec((br, 32), lambda i: (i, 0)),
            pl.BlockSpec((br, n_cls), lambda i: (i, 0)),
        ],
        out_shape=[
            jax.ShapeDtypeStruct((nr, 32), jnp.float32),
            jax.ShapeDtypeStruct((nr, n_cls), jnp.float32),
        ],
        interpret=INTERPRET,
    )(x, mean, std, cs, z, w_bot, b, w1, b1, w2, b2)


def _masked_min_sum(d, m):
    rows = d.shape[0]

    def body(_, carry):
        dcur, s, rem = carry
        v = jnp.min(dcur, axis=1, keepdims=True)
        eq = dcur == v
        cnt = jnp.sum(eq.astype(jnp.float32), axis=1, keepdims=True)
        take = jnp.minimum(cnt, rem)
        s = s + take * v
        rem = rem - take
        dcur = jnp.where(eq, BIG, dcur)
        return dcur, s, rem

    s0 = jnp.zeros((rows, 1), jnp.float32)
    rem0 = jnp.full((rows, 1), float(m), jnp.float32)
    _, s, _ = lax.fori_loop(0, m, body, (d, s0, rem0))
    return s


def _bitonic_sort128(v, payload=None):
    rows, n = v.shape
    lane = lax.broadcasted_iota(jnp.int32, (rows, n), 1)
    size = 2
    while size <= n:
        stride = size // 2
        while stride >= 1:
            pidx = lane ^ stride
            pv = jnp.take_along_axis(v, pidx, axis=1)
            is_lower = (lane & stride) == 0
            want_asc = (lane & size) == 0
            choose_min = is_lower == want_asc
            if payload is not None:
                pp = jnp.take_along_axis(payload, pidx, axis=1)
                keep_own = (choose_min & (v <= pv)) | (
                    (~choose_min) & (v >= pv))
                payload = jnp.where(keep_own, payload, pp)
            v = jnp.where(choose_min, jnp.minimum(v, pv), jnp.maximum(v, pv))
            stride //= 2
        size *= 2
    if payload is None:
        return v
    return v, payload


def _select_groups(d, ng, nsel):
    rows, c = d.shape
    gs = c // ng
    tiles = [d[:, e * ng:(e + 1) * ng] for e in range(gs)]
    while len(tiles) > 1:
        tiles = [jnp.minimum(tiles[i], tiles[i + 1]) if i + 1 < len(tiles)
                 else tiles[i] for i in range(0, len(tiles), 2)]
    gmin = tiles[0]
    lane = lax.broadcasted_iota(jnp.int32, (rows, ng), 1)
    bits = lax.bitcast_convert_type(gmin, jnp.int32)
    keyed = lax.bitcast_convert_type((bits & ~(ng - 1)) | lane, jnp.float32)
    skeys = _bitonic_sort128(keyed)
    idxs = lax.bitcast_convert_type(skeys[:, :nsel], jnp.int32) & (ng - 1)
    parts = [jnp.take_along_axis(d[:, e * ng:(e + 1) * ng], idxs, axis=1)
             for e in range(gs)]
    return jnp.concatenate(parts, axis=1)


def _half_sums(sq):
    rows, c = sq.shape
    ng = 128
    nsel = 32
    if c < 4 * ng or nsel < K + 1:
        d = jnp.sqrt(sq)
        min0 = jnp.min(d, axis=1, keepdims=True)
        return (_masked_min_sum(d, K), _masked_min_sum(d, K + 1), min0)
    cand = _select_groups(sq, ng, nsel)
    cand = _select_groups(cand, ng, nsel)
    svals = jnp.sqrt(_bitonic_sort128(cand))
    lane = lax.broadcasted_iota(jnp.int32, (rows, ng), 1)
    s30 = jnp.sum(jnp.where(lane < K, svals, 0.0), axis=1, keepdims=True)
    s31 = s30 + svals[:, K:K + 1]
    min0 = svals[:, 0:1]
    return s30, s31, min0


def _knn_body(emb_ref, loss_ref):
    p = pl.program_id(0)
    br = KNN_BR
    emb = emb_ref[...]
    sqn = jnp.sum(emb * emb, axis=1)[None, :]
    r0 = p * br
    rows = emb_ref[pl.ds(r0, br), :]
    sqr = jnp.sum(rows * rows, axis=1, keepdims=True)
    g = jax.lax.dot_general(rows, emb, (((1,), (1,)), ((), ())),
                            preferred_element_type=jnp.float32)
    sq = jnp.maximum(sqr + sqn - 2.0 * g, 1e-12)
    s30_0, s31_0, min_0 = _half_sums(sq[:, :NA])
    s30_1, s31_1, min_1 = _half_sums(sq[:, NA:])
    is_a = (r0 < NA)
    s_same = jnp.where(is_a, s31_0, s31_1)
    min0 = jnp.where(is_a, min_0, min_1)
    s_cross = jnp.where(is_a, s30_1, s30_0)
    mean_same = (s_same - min0) / float(K)
    mean_cross = s_cross / float(K)
    mean_same = jnp.where(mean_same == 0.0, 1.0, mean_same)
    mean_cross = jnp.where(mean_cross == 0.0, 1.0, mean_cross)
    ratio = mean_cross / mean_same
    part = jnp.sum(ratio, axis=0, keepdims=True) / float(NA + NB)

    prev = jnp.where(p == 0, jnp.zeros((1, 1), jnp.float32), loss_ref[...])
    val = prev + part
    nprog = pl.num_programs(0)
    val = jnp.where(p == nprog - 1, jnp.maximum(val, 0.0), val)
    loss_ref[...] = val


def _knn_call(all_emb):
    n = all_emb.shape[0]
    br = KNN_BR
    grid = (n // br,)
    return pl.pallas_call(
        _knn_body,
        grid=grid,
        in_specs=[pl.BlockSpec((n, 32), lambda i: (0, 0))],
        out_specs=pl.BlockSpec((1, 1), lambda i: (0, 0)),
        out_shape=jax.ShapeDtypeStruct((1, 1), jnp.float32),
        interpret=INTERPRET,
    )(all_emb)


def kernel(x_init_A, x_init_B, mean_A, mean_B, std_A, std_B,
           gene_matches_A, gene_matches_B, embed_W, embed_b,
           clsA_W1, clsA_b1, clsA_W2, clsA_b2,
           clsB_W1, clsB_b1, clsB_W2, clsB_b2):
    n_a = gene_matches_A.shape[0]
    w_top = embed_W[:n_a]
    w_bot = embed_W[n_a:]

    w_a_eff = _wa_call(gene_matches_A, w_top, w_bot)
    z_b, cs_b = _wb_call(gene_matches_B, w_top)

    x_a, preds_a = _embA_call(x_init_A, mean_A, std_A, w_a_eff, embed_b,
                              clsA_W1, clsA_b1, clsA_W2, clsA_b2,
                              clsA_W2.shape[1])
    x_b, preds_b = _embB_call(x_init_B, mean_B, std_B, cs_b, z_b, w_bot,
                              embed_b, clsB_W1, clsB_b1, clsB_W2, clsB_b2,
                              clsB_W2.shape[1])

    all_emb = jnp.concatenate([x_a, x_b], axis=0)
    loss = _knn_call(all_emb)[0, 0]
    return (preds_a, preds_b, x_a, x_b, loss)


# speedup vs baseline: 1.1652x; 1.1652x over previous
import functools
import jax
import jax.numpy as jnp
from jax import lax
from jax.experimental import pallas as pl
from jax.experimental.pallas import tpu as pltpu

NA = 2048
NB = 2048
K = 30
BIG = 3.0e38

INTERPRET = False
WA_BR = 1000
EMB_BR = 512
KNN_BR = 512


def _wa_body(gm_ref, wtop_ref, wbot_ref, out_ref):
    gm = gm_ref[...]
    rs = jnp.sum(gm, axis=1, keepdims=True)
    rs = jnp.where(rs == 0.0, 1.0, rs)
    acc = jax.lax.dot_general(gm, wbot_ref[...], (((1,), (0,)), ((), ())),
                              preferred_element_type=jnp.float32)
    out_ref[...] = wtop_ref[...] + acc / rs


def _wa_call(gm_a, w_top, w_bot):
    n, m = gm_a.shape
    br = WA_BR
    grid = (n // br,)
    return pl.pallas_call(
        _wa_body,
        grid=grid,
        in_specs=[
            pl.BlockSpec((br, m), lambda i: (i, 0)),
            pl.BlockSpec((br, 32), lambda i: (i, 0)),
            pl.BlockSpec((m, 32), lambda i: (0, 0)),
        ],
        out_specs=pl.BlockSpec((br, 32), lambda i: (i, 0)),
        out_shape=jax.ShapeDtypeStruct((n, 32), jnp.float32),
        interpret=INTERPRET,
    )(gm_a, w_top, w_bot)


def _wb_body(gm_ref, wtop_ref, z_ref, cs_ref):
    p = pl.program_id(0)
    gm = gm_ref[...]
    z_part = jax.lax.dot_general(gm, wtop_ref[...], (((0,), (0,)), ((), ())),
                                 preferred_element_type=jnp.float32)
    cs_part = jnp.sum(gm, axis=0, keepdims=True)

    @pl.when(p == 0)
    def _():
        z_ref[...] = z_part
        cs_ref[...] = cs_part

    @pl.when(p != 0)
    def _():
        z_ref[...] = z_ref[...] + z_part
        cs_ref[...] = cs_ref[...] + cs_part


def _wb_call(gm_b, w_top):
    n, m = gm_b.shape
    br = WA_BR
    grid = (n // br,)
    return pl.pallas_call(
        _wb_body,
        grid=grid,
        in_specs=[
            pl.BlockSpec((br, m), lambda i: (i, 0)),
            pl.BlockSpec((br, 32), lambda i: (i, 0)),
        ],
        out_specs=[
            pl.BlockSpec((m, 32), lambda i: (0, 0)),
            pl.BlockSpec((1, m), lambda i: (0, 0)),
        ],
        out_shape=[
            jax.ShapeDtypeStruct((m, 32), jnp.float32),
            jax.ShapeDtypeStruct((1, m), jnp.float32),
        ],
        interpret=INTERPRET,
    )(gm_b, w_top)


def _cls_tail(e, w1_ref, b1_ref, w2_ref, b2_ref):
    h = jax.nn.relu(jnp.dot(e, w1_ref[...], preferred_element_type=jnp.float32)
                    + b1_ref[...][None, :])
    logit = jnp.dot(h, w2_ref[...], preferred_element_type=jnp.float32) \
        + b2_ref[...][None, :]
    mx = jnp.max(logit, axis=1, keepdims=True)
    ex = jnp.exp(logit - mx)
    return ex / jnp.sum(ex, axis=1, keepdims=True)


def _embA_body(x_ref, mean_ref, std_ref, w_ref, b_ref,
               w1_ref, b1_ref, w2_ref, b2_ref, xa_ref, pred_ref):
    xn = (x_ref[...] - mean_ref[...][None, :]) / std_ref[...][None, :]
    e = jax.nn.relu(jnp.dot(xn, w_ref[...], preferred_element_type=jnp.float32)
                    + b_ref[...][None, :])
    xa_ref[...] = e
    pred_ref[...] = _cls_tail(e, w1_ref, b1_ref, w2_ref, b2_ref)


def _embA_call(x, mean, std, w_eff, b, w1, b1, w2, b2, n_cls):
    nr, d = x.shape
    br = EMB_BR
    grid = (nr // br,)
    return pl.pallas_call(
        _embA_body,
        grid=grid,
        in_specs=[
            pl.BlockSpec((br, d), lambda i: (i, 0)),
            pl.BlockSpec((d,), lambda i: (0,)),
            pl.BlockSpec((d,), lambda i: (0,)),
            pl.BlockSpec((d, 32), lambda i: (0, 0)),
            pl.BlockSpec((32,), lambda i: (0,)),
            pl.BlockSpec((32, 16), lambda i: (0, 0)),
            pl.BlockSpec((16,), lambda i: (0,)),
            pl.BlockSpec((16, n_cls), lambda i: (0, 0)),
            pl.BlockSpec((n_cls,), lambda i: (0,)),
        ],
        out_specs=[
            pl.BlockSpec((br, 32), lambda i: (i, 0)),
            pl.BlockSpec((br, n_cls), lambda i: (i, 0)),
        ],
        out_shape=[
            jax.ShapeDtypeStruct((nr, 32), jnp.float32),
            jax.ShapeDtypeStruct((nr, n_cls), jnp.float32),
        ],
        interpret=INTERPRET,
    )(x, mean, std, w_eff, b, w1, b1, w2, b2)


def _embB_body(x_ref, mean_ref, std_ref, cs_ref, z_ref, wbot_ref, b_ref,
               w1_ref, b1_ref, w2_ref, b2_ref, xb_ref, pred_ref):
    xn = (x_ref[...] - mean_ref[...][None, :]) / std_ref[...][None, :]
    cs = cs_ref[...]
    cs = jnp.where(cs == 0.0, 1.0, cs)
    xs = xn / cs
    e = jnp.dot(xs, z_ref[...], preferred_element_type=jnp.float32) \
        + jnp.dot(xn, wbot_ref[...], preferred_element_type=jnp.float32) \
        + b_ref[...][None, :]
    e = jax.nn.relu(e)
    xb_ref[...] = e
    pred_ref[...] = _cls_tail(e, w1_ref, b1_ref, w2_ref, b2_ref)


def _embB_call(x, mean, std, cs, z, w_bot, b, w1, b1, w2, b2, n_cls):
    nr, d = x.shape
    br = EMB_BR
    grid = (nr // br,)
    return pl.pallas_call(
        _embB_body,
        grid=grid,
        in_specs=[
            pl.BlockSpec((br, d), lambda i: (i, 0)),
            pl.BlockSpec((d,), lambda i: (0,)),
            pl.BlockSpec((d,), lambda i: (0,)),
            pl.BlockSpec((1, d), lambda i: (0, 0)),
            pl.BlockSpec((d, 32), lambda i: (0, 0)),
            pl.BlockSpec((d, 32), lambda i: (0, 0)),
            pl.BlockSpec((32,), lambda i: (0,)),
            pl.BlockSpec((32, 16), lambda i: (0, 0)),
            pl.BlockSpec((16,), lambda i: (0,)),
            pl.BlockSpec((16, n_cls), lambda i: (0, 0)),
            pl.BlockSpec((n_cls,), lambda i: (0,)),
        ],
        out_specs=[
            pl.BlockSpec((br, 32), lambda i: (i, 0)),
            pl.BlockSpec((br, n_cls), lambda i: (i, 0)),
        ],
        out_shape=[
            jax.ShapeDtypeStruct((nr, 32), jnp.float32),
            jax.ShapeDtypeStruct((nr, n_cls), jnp.float32),
        ],
        interpret=INTERPRET,
    )(x, mean, std, cs, z, w_bot, b, w1, b1, w2, b2)


def _masked_min_sum(d, m):
    rows = d.shape[0]

    def body(_, carry):
        dcur, s, rem = carry
        v = jnp.min(dcur, axis=1, keepdims=True)
        eq = dcur == v
        cnt = jnp.sum(eq.astype(jnp.float32), axis=1, keepdims=True)
        take = jnp.minimum(cnt, rem)
        s = s + take * v
        rem = rem - take
        dcur = jnp.where(eq, BIG, dcur)
        return dcur, s, rem

    s0 = jnp.zeros((rows, 1), jnp.float32)
    rem0 = jnp.full((rows, 1), float(m), jnp.float32)
    _, s, _ = lax.fori_loop(0, m, body, (d, s0, rem0))
    return s


def _bitonic_sort128(v, payload=None):
    rows, n = v.shape
    lane = lax.broadcasted_iota(jnp.int32, (rows, n), 1)
    size = 2
    while size <= n:
        stride = size // 2
        while stride >= 1:
            pidx = lane ^ stride
            pv = jnp.take_along_axis(v, pidx, axis=1)
            is_lower = (lane & stride) == 0
            want_asc = (lane & size) == 0
            choose_min = is_lower == want_asc
            if payload is not None:
                pp = jnp.take_along_axis(payload, pidx, axis=1)
                keep_own = (choose_min & (v <= pv)) | (
                    (~choose_min) & (v >= pv))
                payload = jnp.where(keep_own, payload, pp)
            v = jnp.where(choose_min, jnp.minimum(v, pv), jnp.maximum(v, pv))
            stride //= 2
        size *= 2
    if payload is None:
        return v
    return v, payload


def _select_groups(d, ng, nsel):
    rows, c = d.shape
    gs = c // ng
    tiles = [d[:, e * ng:(e + 1) * ng] for e in range(gs)]
    while len(tiles) > 1:
        tiles = [jnp.minimum(tiles[i], tiles[i + 1]) if i + 1 < len(tiles)
                 else tiles[i] for i in range(0, len(tiles), 2)]
    gmin = tiles[0]
    lane = lax.broadcasted_iota(jnp.int32, (rows, ng), 1)
    bits = lax.bitcast_convert_type(gmin, jnp.int32)
    keyed = lax.bitcast_convert_type((bits & ~(ng - 1)) | lane, jnp.float32)
    skeys = _bitonic_sort128(keyed)
    idxs = lax.bitcast_convert_type(skeys[:, :nsel], jnp.int32) & (ng - 1)
    parts = [jnp.take_along_axis(d[:, e * ng:(e + 1) * ng], idxs, axis=1)
             for e in range(gs)]
    return jnp.concatenate(parts, axis=1)


def _half_sums(sq):
    rows, c = sq.shape
    ng = 128
    nsel = 32
    if c < 4 * ng or nsel < K + 1:
        d = jnp.sqrt(sq)
        min0 = jnp.min(d, axis=1, keepdims=True)
        return (_masked_min_sum(d, K), _masked_min_sum(d, K + 1), min0)
    cand = _select_groups(sq, ng, nsel)
    cand = _select_groups(cand, ng, nsel)
    svals = jnp.sqrt(_bitonic_sort128(cand))
    lane = lax.broadcasted_iota(jnp.int32, (rows, ng), 1)
    s30 = jnp.sum(jnp.where(lane < K, svals, 0.0), axis=1, keepdims=True)
    s31 = s30 + svals[:, K:K + 1]
    min0 = svals[:, 0:1]
    return s30, s31, min0


def _knn_body(emb_ref, loss_ref):
    p = pl.program_id(0)
    br = KNN_BR
    emb = emb_ref[...]
    sqn = jnp.sum(emb * emb, axis=1)[None, :]
    r0 = p * br
    rows = emb_ref[pl.ds(r0, br), :]
    sqr = jnp.sum(rows * rows, axis=1, keepdims=True)
    g = jax.lax.dot_general(rows, emb, (((1,), (1,)), ((), ())),
                            preferred_element_type=jnp.float32)
    sq = jnp.maximum(sqr + sqn - 2.0 * g, 1e-12)
    s30_0, s31_0, min_0 = _half_sums(sq[:, :NA])
    s30_1, s31_1, min_1 = _half_sums(sq[:, NA:])
    is_a = (r0 < NA)
    s_same = jnp.where(is_a, s31_0, s31_1)
    min0 = jnp.where(is_a, min_0, min_1)
    s_cross = jnp.where(is_a, s30_1, s30_0)
    mean_same = (s_same - min0) / float(K)
    mean_cross = s_cross / float(K)
    mean_same = jnp.where(mean_same == 0.0, 1.0, mean_same)
    mean_cross = jnp.where(mean_cross == 0.0, 1.0, mean_cross)
    ratio = mean_cross / mean_same
    part = jnp.sum(ratio, axis=0, keepdims=True) / float(NA + NB)

    prev = jnp.where(p == 0, jnp.zeros((1, 1), jnp.float32), loss_ref[...])
    val = prev + part
    nprog = pl.num_programs(0)
    val = jnp.where(p == nprog - 1, jnp.maximum(val, 0.0), val)
    loss_ref[...] = val


def _knn_call(all_emb):
    n = all_emb.shape[0]
    br = KNN_BR
    grid = (n // br,)
    return pl.pallas_call(
        _knn_body,
        grid=grid,
        in_specs=[pl.BlockSpec((n, 32), lambda i: (0, 0))],
        out_specs=pl.BlockSpec((1, 1), lambda i: (0, 0)),
        out_shape=jax.ShapeDtypeStruct((1, 1), jnp.float32),
        interpret=INTERPRET,
    )(all_emb)


def kernel(x_init_A, x_init_B, mean_A, mean_B, std_A, std_B,
           gene_matches_A, gene_matches_B, embed_W, embed_b,
           clsA_W1, clsA_b1, clsA_W2, clsA_b2,
           clsB_W1, clsB_b1, clsB_W2, clsB_b2):
    n_a = gene_matches_A.shape[0]
    w_top = embed_W[:n_a]
    w_bot = embed_W[n_a:]

    w_a_eff = _wa_call(gene_matches_A, w_top, w_bot)
    z_b, cs_b = _wb_call(gene_matches_B, w_top)

    x_a, preds_a = _embA_call(x_init_A, mean_A, std_A, w_a_eff, embed_b,
                              clsA_W1, clsA_b1, clsA_W2, clsA_b2,
                              clsA_W2.shape[1])
    x_b, preds_b = _embB_call(x_init_B, mean_B, std_B, cs_b, z_b, w_bot,
                              embed_b, clsB_W1, clsB_b1, clsB_W2, clsB_b2,
                              clsB_W2.shape[1])

    all_emb = jnp.concatenate([x_a, x_b], axis=0)
    loss = _knn_call(all_emb)[0, 0]
    return (preds_a, preds_b, x_a, x_b, loss)
